# Initial kernel scaffold; baseline (speedup 1.0000x reference)
import jax
import jax.numpy as jnp
from jax.experimental import pallas as pl


def kernel(x, table):
    raise NotImplementedError("write your pallas kernel here")


# speedup vs baseline: 139.2777x; 139.2777x over previous
import functools

import jax
import jax.numpy as jnp
from jax import lax
from jax.experimental import pallas as pl
from jax.experimental.pallas import tpu as pltpu
from jax.experimental.pallas import tpu_sc as plsc

_BATCH = 16384
_HIST = 200
_N = _BATCH * _HIST
_NW = 32
_PER_W = _N // _NW
_CHUNK = 25600
_NCHUNK = _PER_W // _CHUNK

_mesh = plsc.VectorSubcoreMesh(core_axis_name="c", subcore_axis_name="s")


@functools.partial(
    pl.kernel,
    mesh=_mesh,
    out_type=jax.ShapeDtypeStruct((_N,), jnp.int32),
    scratch_types=[
        pltpu.VMEM((_CHUNK,), jnp.int32),
        pltpu.VMEM((_CHUNK,), jnp.int32),
        pltpu.SemaphoreType.DMA,
    ],
)
def _lookup(x_hbm, table_hbm, out_hbm, idx_v, vals_v, sem):
    wid = lax.axis_index("s") * 2 + lax.axis_index("c")
    base = wid * _PER_W

    def body(i, carry):
        off = base + i * _CHUNK
        pltpu.sync_copy(x_hbm.at[pl.ds(off, _CHUNK)], idx_v)
        pltpu.async_copy(table_hbm.at[idx_v], vals_v, sem).wait()
        pltpu.sync_copy(vals_v, out_hbm.at[pl.ds(off, _CHUNK)])
        return carry

    lax.fori_loop(0, _NCHUNK, body, 0)


def kernel(x, table):
    out = _lookup(x.reshape(_N), table)
    return out.reshape(x.shape)


# speedup vs baseline: 140.1222x; 1.0061x over previous
import functools

import jax
import jax.numpy as jnp
from jax import lax
from jax.experimental import pallas as pl
from jax.experimental.pallas import tpu as pltpu
from jax.experimental.pallas import tpu_sc as plsc

_BATCH = 16384
_HIST = 200
_N = _BATCH * _HIST
_NW = 32
_PER_W = _N // _NW
_CHUNK = 12800
_NCHUNK = _PER_W // _CHUNK
_NBUF = 3

_mesh = plsc.VectorSubcoreMesh(core_axis_name="c", subcore_axis_name="s")


@functools.partial(
    pl.kernel,
    mesh=_mesh,
    out_type=jax.ShapeDtypeStruct((_N,), jnp.int32),
    scratch_types=(
        [pltpu.VMEM((_CHUNK,), jnp.int32) for _ in range(2 * _NBUF)]
        + [pltpu.SemaphoreType.DMA((_NBUF,)) for _ in range(3)]
    ),
)
def _lookup(x_hbm, table_hbm, out_hbm, i0, i1, i2, v0, v1, v2,
            sem_i, sem_g, sem_w):
    idx_v = [i0, i1, i2]
    vals_v = [v0, v1, v2]
    wid = lax.axis_index("s") * 2 + lax.axis_index("c")
    base = wid * _PER_W

    def idx_load(g):
        b = g % _NBUF
        return pltpu.async_copy(
            x_hbm.at[pl.ds(base + g * _CHUNK, _CHUNK)], idx_v[b], sem_i.at[b])

    def gather(g):
        b = g % _NBUF
        return pltpu.async_copy(table_hbm.at[idx_v[b]], vals_v[b],
                                sem_g.at[b])

    def writeback(g):
        b = g % _NBUF
        return pltpu.async_copy(
            vals_v[b], out_hbm.at[pl.ds(base + g * _CHUNK, _CHUNK)],
            sem_w.at[b])

    h_i = {}
    h_g = {}
    h_w = {}
    for g in range(_NBUF):
        h_i[g] = idx_load(g)
    for g in range(_NCHUNK):
        h_i[g].wait()
        if g >= _NBUF:
            h_w[g - _NBUF].wait()
        h_g[g] = gather(g)
        if g >= 1:
            h_g[g - 1].wait()
            h_w[g - 1] = writeback(g - 1)
            if g + _NBUF - 1 < _NCHUNK:
                h_i[g + _NBUF - 1] = idx_load(g + _NBUF - 1)
    h_g[_NCHUNK - 1].wait()
    h_w[_NCHUNK - 1] = writeback(_NCHUNK - 1)
    for g in range(_NCHUNK - _NBUF, _NCHUNK):
        h_w[g].wait()


def kernel(x, table):
    out = _lookup(x.reshape(_N), table)
    return out.reshape(x.shape)


# speedup vs baseline: 220.1797x; 1.5713x over previous
import functools

import jax
import jax.numpy as jnp
from jax import lax
from jax.experimental import pallas as pl
from jax.experimental.pallas import tpu as pltpu
from jax.experimental.pallas import tpu_sc as plsc

_VOCAB = 1000000
_BATCH = 16384
_HIST = 200
_N = _BATCH * _HIST
_NW = 32
_PER_W = _N // _NW
_CHUNK = 10240
_NCHUNK = _PER_W // _CHUNK
_NBUF = 3
_STAGERS = 10
_STAGE = _VOCAB // _STAGERS
_BOUNCE = 10000
_NSTAGE = _STAGE // _BOUNCE

_mesh = plsc.VectorSubcoreMesh(core_axis_name="c", subcore_axis_name="s")


@functools.partial(
    pl.kernel,
    mesh=_mesh,
    out_type=jax.ShapeDtypeStruct((_N,), jnp.int32),
    scratch_types=(
        [pltpu.VMEM_SHARED((_VOCAB,), jnp.int32)]
        + [pltpu.VMEM((_CHUNK,), jnp.int32) for _ in range(2 * _NBUF)]
        + [pltpu.SemaphoreType.DMA((_NBUF,)) for _ in range(3)]
    ),
)
def _lookup(x_hbm, table_hbm, out_hbm, table_sp, i0, i1, i2, v0, v1, v2,
            sem_i, sem_g, sem_w):
    idx_v = [i0, i1, i2]
    vals_v = [v0, v1, v2]
    s = lax.axis_index("s")
    wid = s * 2 + lax.axis_index("c")
    base = wid * _PER_W

    @pl.when(s < _STAGERS)
    def _():
        for j in range(_NSTAGE):
            off = s * _STAGE + j * _BOUNCE
            pltpu.sync_copy(table_hbm.at[pl.ds(off, _BOUNCE)],
                            i0.at[pl.ds(0, _BOUNCE)])
            pltpu.sync_copy(i0.at[pl.ds(0, _BOUNCE)],
                            table_sp.at[pl.ds(off, _BOUNCE)])

    plsc.subcore_barrier()

    def idx_load(g):
        b = g % _NBUF
        return pltpu.async_copy(
            x_hbm.at[pl.ds(base + g * _CHUNK, _CHUNK)], idx_v[b], sem_i.at[b])

    def gather(g):
        b = g % _NBUF
        return pltpu.async_copy(table_sp.at[idx_v[b]], vals_v[b],
                                sem_g.at[b])

    def writeback(g):
        b = g % _NBUF
        return pltpu.async_copy(
            vals_v[b], out_hbm.at[pl.ds(base + g * _CHUNK, _CHUNK)],
            sem_w.at[b])

    h_i = {}
    h_g = {}
    h_w = {}
    for g in range(_NBUF):
        h_i[g] = idx_load(g)
    for g in range(_NCHUNK):
        h_i[g].wait()
        if g >= _NBUF:
            h_w[g - _NBUF].wait()
        h_g[g] = gather(g)
        if g >= 1:
            h_g[g - 1].wait()
            h_w[g - 1] = writeback(g - 1)
            if g + _NBUF - 1 < _NCHUNK:
                h_i[g + _NBUF - 1] = idx_load(g + _NBUF - 1)
    h_g[_NCHUNK - 1].wait()
    h_w[_NCHUNK - 1] = writeback(_NCHUNK - 1)
    for g in range(_NCHUNK - _NBUF, _NCHUNK):
        h_w[g].wait()


def kernel(x, table):
    out = _lookup(x.reshape(_N), table)
    return out.reshape(x.shape)
